# baseline (speedup 1.0000x reference)
import functools

import jax
import jax.numpy as jnp
from jax import lax
from jax.experimental import pallas as pl
from jax.experimental.pallas import tpu as pltpu
from jax.experimental.pallas import tpu_sc as plsc

N = 10000
E = 320000
D = 128

NC = 2
NS = 16
L = 16
NW = NC * NS
EPW = E // NW
K = 80
NCHUNK = EPW // K
NP = 10240
RPT = NP // NS
ZR = 128
WC = 40

BM = 1000
NB = N // BM


def _mm_body(x_ref, w_ref, sup_ref):
    sup_ref[...] = jnp.dot(
        x_ref[...], w_ref[...], preferred_element_type=jnp.float32
    )


def _support_mm(x, weight):
    return pl.pallas_call(
        _mm_body,
        grid=(NB,),
        in_specs=[
            pl.BlockSpec((BM, D), lambda i: (i, 0)),
            pl.BlockSpec((D, D), lambda i: (0, 0)),
        ],
        out_specs=pl.BlockSpec((BM, D), lambda i: (i, 0)),
        out_shape=jax.ShapeDtypeStruct((N, D), jnp.float32),
    )(x, weight)


def _agg_body(sup_hbm, packed_hbm, w_hbm, out_hbm,
              packed_all, w_stage,
              idx0, idx1, idx2, rows0, rows1, rows2,
              acc_sh, gsem0, gsem1, gsem2, ssem0, ssem1, ssem2):
    idx = (idx0, idx1, idx2)
    rows = (rows0, rows1, rows2)
    gsem = (gsem0, gsem1, gsem2)
    ssem = (ssem0, ssem1, ssem2)
    cid = lax.axis_index("c")
    sid = lax.axis_index("s")
    wid = sid * NC + cid

    pltpu.sync_copy(packed_hbm.at[wid], packed_all)
    pltpu.sync_copy(w_hbm.at[wid], w_stage)

    def _zrow(r, _):
        for j in range(D // L):
            rows0[r, pl.ds(j * L, L)] = jnp.zeros((L,), jnp.float32)
        return 0
    lax.fori_loop(0, K, _zrow, 0)
    for k in range(RPT // K):
        pltpu.sync_copy(rows0, acc_sh.at[pl.ds(sid * RPT + k * K, K)])
    plsc.subcore_barrier()

    def _unpack_src(t, b):
        def _g(g, _):
            p = packed_all[pl.ds(t * K + g * L, L)]
            s = jnp.minimum(jnp.bitwise_and(p, 16383), N - 1)
            idx[b][pl.ds(g * L, L)] = s
            return 0
        lax.fori_loop(0, K // L, _g, 0)

    def _unpack_dst(t, b):
        def _g(g, _):
            p = packed_all[pl.ds(t * K + g * L, L)]
            idx[b][pl.ds(g * L, L)] = lax.shift_right_logical(p, 14)
            return 0
        lax.fori_loop(0, K // L, _g, 0)

    def _scale(b, t):
        rv = rows[b]

        def _mul(v, i, e):
            wa = lax.bitcast_convert_type(jnp.left_shift(v, 16), jnp.float32)
            wb = lax.bitcast_convert_type(
                jnp.bitwise_and(v, jnp.int32(-65536)), jnp.float32
            )
            for wv, eo in ((wa, e), (wb, e + 1)):
                wi = jnp.full((L,), wv[i], jnp.float32)
                for j in range(D // L):
                    rv[eo, pl.ds(j * L, L)] = rv[eo, pl.ds(j * L, L)] * wi

        def _pair(h, _):
            v = w_stage[pl.ds(t * WC + L * h, L)]
            for i in range(L):
                _mul(v, i, 32 * h + 2 * i)
            return 0
        lax.fori_loop(0, 2, _pair, 0)
        vt = w_stage[pl.ds(t * WC + 2 * L, L)]
        for i in range(8):
            _mul(vt, i, 64 + 2 * i)

    def _issue_gather(t, b):
        _unpack_src(t, b)
        pltpu.async_copy(sup_hbm.at[idx[b]], rows[b], gsem[b])

    def _wait_gather(b):
        pltpu.make_async_copy(sup_hbm.at[idx[b]], rows[b], gsem[b]).wait()

    def _issue_scatter(t, b):
        _unpack_dst(t, b)
        pltpu.async_copy(rows[b], acc_sh.at[idx[b]], ssem[b], add=True)

    def _wait_scatter(b):
        pltpu.make_async_copy(rows[b], acc_sh.at[idx[b]], ssem[b]).wait()

    _issue_gather(0, 0)
    _issue_gather(1, 1)

    def _steady(u, _):
        for i in range(3):
            c = 3 * u + i
            b = i
            bn = (i + 2) % 3
            _wait_gather(b)
            _scale(b, c)

            @pl.when(c < NCHUNK)
            def _():
                _issue_scatter(c, b)

            @pl.when(c >= 1)
            def _():
                _wait_scatter(bn)
            _issue_gather(c + 2, bn)
        return 0
    lax.fori_loop(0, (NCHUNK + 1) // 3, _steady, 0)
    _wait_gather(0)
    _wait_gather(1)
    plsc.subcore_barrier()

    for k in range(RPT // ZR):
        off = sid * RPT + k * ZR
        pltpu.sync_copy(acc_sh.at[pl.ds(off, ZR)], out_hbm.at[cid, pl.ds(off, ZR)])


def _aggregate(support, src, dst, edge_weight):
    mesh = plsc.VectorSubcoreMesh(core_axis_name="c", subcore_axis_name="s")
    f = functools.partial(
        pl.kernel,
        mesh=mesh,
        out_type=jax.ShapeDtypeStruct((NC, NP, D), jnp.float32),
        scratch_types=[
            pltpu.VMEM((EPW + 3 * K,), jnp.int32),
            pltpu.VMEM((EPW // 2 + K + L,), jnp.int32),
            pltpu.VMEM((K,), jnp.int32),
            pltpu.VMEM((K,), jnp.int32),
            pltpu.VMEM((K,), jnp.int32),
            pltpu.VMEM((K, D), jnp.float32),
            pltpu.VMEM((K, D), jnp.float32),
            pltpu.VMEM((K, D), jnp.float32),
            pltpu.VMEM_SHARED((NP, D), jnp.float32),
            pltpu.SemaphoreType.DMA,
            pltpu.SemaphoreType.DMA,
            pltpu.SemaphoreType.DMA,
            pltpu.SemaphoreType.DMA,
            pltpu.SemaphoreType.DMA,
            pltpu.SemaphoreType.DMA,
        ],
    )(_agg_body)
    packed = jnp.bitwise_or(jnp.left_shift(dst, 14), src).reshape(NW, EPW)
    packed = jnp.pad(packed, ((0, 0), (0, 3 * K)))
    w_st = lax.bitcast_convert_type(
        edge_weight.astype(jnp.bfloat16).reshape(E // 2, 2), jnp.int32
    ).reshape(NW, EPW // 2)
    w_st = jnp.pad(w_st, ((0, 0), (0, K + L)))
    return f(support, packed, w_st)


def _comb_body(p_ref, x_ref, ws_ref, b_ref, pre_ref, st_ref):
    i = pl.program_id(0)
    v = p_ref[0] + p_ref[1] + b_ref[...] + jnp.dot(
        x_ref[...], ws_ref[...], preferred_element_type=jnp.float32
    )
    pre_ref[...] = v
    cs = jnp.sum(v, axis=0, keepdims=True)
    cs2 = jnp.sum(v * v, axis=0, keepdims=True)
    st = jnp.concatenate([cs, cs2, jnp.zeros((6, D), jnp.float32)], axis=0)

    @pl.when(i == 0)
    def _():
        st_ref[...] = st

    @pl.when(i > 0)
    def _():
        st_ref[...] += st


def _combine(partials, x, self_weight, bias):
    return pl.pallas_call(
        _comb_body,
        grid=(NB,),
        in_specs=[
            pl.BlockSpec((NC, BM, D), lambda i: (0, i, 0)),
            pl.BlockSpec((BM, D), lambda i: (i, 0)),
            pl.BlockSpec((D, D), lambda i: (0, 0)),
            pl.BlockSpec((1, D), lambda i: (0, 0)),
        ],
        out_specs=[
            pl.BlockSpec((BM, D), lambda i: (i, 0)),
            pl.BlockSpec((8, D), lambda i: (0, 0)),
        ],
        out_shape=[
            jax.ShapeDtypeStruct((N, D), jnp.float32),
            jax.ShapeDtypeStruct((8, D), jnp.float32),
        ],
    )(partials, x, self_weight, bias.reshape(1, D))


def _bn_body(pre_ref, st_ref, g_ref, b_ref, o_ref):
    s = st_ref[0:1, :]
    s2 = st_ref[1:2, :]
    mean = s / N
    var = s2 / N - mean * mean
    rstd = lax.rsqrt(var + 1e-5)
    o_ref[...] = (pre_ref[...] - mean) * (rstd * g_ref[...]) + b_ref[...]


def _batchnorm(pre, stats, gamma, beta):
    return pl.pallas_call(
        _bn_body,
        grid=(NB,),
        in_specs=[
            pl.BlockSpec((BM, D), lambda i: (i, 0)),
            pl.BlockSpec((8, D), lambda i: (0, 0)),
            pl.BlockSpec((1, D), lambda i: (0, 0)),
            pl.BlockSpec((1, D), lambda i: (0, 0)),
        ],
        out_specs=pl.BlockSpec((BM, D), lambda i: (i, 0)),
        out_shape=jax.ShapeDtypeStruct((N, D), jnp.float32),
    )(pre, stats, gamma.reshape(1, D), beta.reshape(1, D))


def kernel(x, edge_weight, weight, self_weight, bias, gamma, beta, edge_index):
    support = _support_mm(x, weight)
    dst = edge_index[0]
    src = edge_index[1]
    partials = _aggregate(support, src, dst, edge_weight)
    pre, stats = _combine(partials, x, self_weight, bias)
    return _batchnorm(pre, stats, gamma, beta)


# speedup vs baseline: 3.2506x; 3.2506x over previous
import functools

import jax
import jax.numpy as jnp
from jax import lax
from jax.experimental import pallas as pl
from jax.experimental.pallas import tpu as pltpu
from jax.experimental.pallas import tpu_sc as plsc

N = 10000
E = 320000
D = 128

NC = 2
NS = 16
L = 16
NW = NC * NS
EPW = E // NW
K = 80
NCHUNK = EPW // K
NP = 10240
RPT = NP // NS
ZR = 128
WC = 40
WPAD = 8

BM = 1000
NB = N // BM


def _mm_body(x_ref, w_ref, sup_ref):
    sup_ref[...] = jnp.dot(
        x_ref[...], w_ref[...], preferred_element_type=jnp.float32
    )


def _support_mm(x, weight):
    return pl.pallas_call(
        _mm_body,
        grid=(NB,),
        in_specs=[
            pl.BlockSpec((BM, D), lambda i: (i, 0)),
            pl.BlockSpec((D, D), lambda i: (0, 0)),
        ],
        out_specs=pl.BlockSpec((BM, D), lambda i: (i, 0)),
        out_shape=jax.ShapeDtypeStruct((N, D), jnp.float32),
    )(x, weight)


def _agg_body(sup_hbm, packed_hbm, w_hbm, out_hbm,
              packed_all, w_stage, w_chunk,
              idx0, idx1, idx2, rows0, rows1, rows2,
              acc_sh, gsem0, gsem1, gsem2, ssem0, ssem1, ssem2):
    idx = (idx0, idx1, idx2)
    rows = (rows0, rows1, rows2)
    gsem = (gsem0, gsem1, gsem2)
    ssem = (ssem0, ssem1, ssem2)
    cid = lax.axis_index("c")
    sid = lax.axis_index("s")
    wid = sid * NC + cid

    pltpu.sync_copy(packed_hbm.at[wid], packed_all)
    pltpu.sync_copy(w_hbm.at[wid], w_stage)

    def _zrow(r, _):
        for j in range(D // L):
            rows0[r, pl.ds(j * L, L)] = jnp.zeros((L,), jnp.float32)
        return 0
    lax.fori_loop(0, K, _zrow, 0)
    for k in range(RPT // K):
        pltpu.sync_copy(rows0, acc_sh.at[pl.ds(sid * RPT + k * K, K)])
    plsc.subcore_barrier()

    def _unpack_src(t, b):
        def _g(g, _):
            p = packed_all[pl.ds(t * K + g * L, L)]
            idx[b][pl.ds(g * L, L)] = jnp.bitwise_and(p, 16383)
            return 0
        lax.fori_loop(0, K // L, _g, 0)

    def _unpack_dst(t, b):
        def _g(g, _):
            p = packed_all[pl.ds(t * K + g * L, L)]
            idx[b][pl.ds(g * L, L)] = lax.shift_right_logical(p, 14)
            return 0
        lax.fori_loop(0, K // L, _g, 0)

    def _expand(v):
        wa = lax.bitcast_convert_type(jnp.left_shift(v, 16), jnp.float32)
        wb = lax.bitcast_convert_type(
            jnp.bitwise_and(v, jnp.int32(-65536)), jnp.float32
        )
        return wa, wb

    def _unpack_w(t):
        vt = w_stage[pl.ds(t * WC + 2 * L, L)]
        wa, wb = _expand(vt)
        w_chunk[pl.ds(2 * L, L)] = wa
        w_chunk[pl.ds(72, L)] = wb

        def _g(g, _):
            v = w_stage[pl.ds(t * WC + g * L, L)]
            wa, wb = _expand(v)
            w_chunk[pl.ds(g * L, L)] = wa
            w_chunk[pl.ds(40 + g * L, L)] = wb
            return 0
        lax.fori_loop(0, 2, _g, 0)

    def _scale(b):
        rv = rows[b]

        def _body(g, _):
            w16 = w_chunk[pl.ds(g * L, L)]
            for i in range(L):
                wi = jnp.full((L,), w16[i], jnp.float32)
                e = g * L + i
                for j in range(D // L):
                    rv[e, pl.ds(j * L, L)] = rv[e, pl.ds(j * L, L)] * wi
            return 0
        lax.fori_loop(0, K // L, _body, 0)

    def _issue_gather(t, b):
        _unpack_src(t, b)
        pltpu.async_copy(sup_hbm.at[idx[b]], rows[b], gsem[b])

    def _wait_gather(b):
        pltpu.make_async_copy(sup_hbm.at[idx[b]], rows[b], gsem[b]).wait()

    def _issue_scatter(t, b):
        _unpack_dst(t, b)
        pltpu.async_copy(rows[b], acc_sh.at[idx[b]], ssem[b], add=True)

    def _wait_scatter(b):
        pltpu.make_async_copy(rows[b], acc_sh.at[idx[b]], ssem[b]).wait()

    _issue_gather(0, 0)
    _issue_gather(1, 1)

    def _steady(u, _):
        for i in range(3):
            c = 3 * u + i
            b = i
            bn = (i + 2) % 3
            _wait_gather(b)
            _unpack_w(c)
            _scale(b)
            _issue_scatter(c, b)

            @pl.when(c >= 1)
            def _():
                _wait_scatter(bn)
            _issue_gather(c + 2, bn)
        return 0
    lax.fori_loop(0, (NCHUNK - 2) // 3, _steady, 0)

    for c in (NCHUNK - 2, NCHUNK - 1):
        b = c % 3
        _wait_gather(b)
        _unpack_w(c)
        _scale(b)
        _issue_scatter(c, b)
    for b in ((NCHUNK - 3) % 3, (NCHUNK - 2) % 3, (NCHUNK - 1) % 3):
        _wait_scatter(b)
    plsc.subcore_barrier()

    for k in range(RPT // ZR):
        off = sid * RPT + k * ZR
        pltpu.sync_copy(acc_sh.at[pl.ds(off, ZR)], out_hbm.at[cid, pl.ds(off, ZR)])


def _aggregate(support, src, dst, edge_weight):
    mesh = plsc.VectorSubcoreMesh(core_axis_name="c", subcore_axis_name="s")
    f = functools.partial(
        pl.kernel,
        mesh=mesh,
        out_type=jax.ShapeDtypeStruct((NC, NP, D), jnp.float32),
        scratch_types=[
            pltpu.VMEM((EPW,), jnp.int32),
            pltpu.VMEM((NCHUNK * WC + WPAD,), jnp.int32),
            pltpu.VMEM((96,), jnp.float32),
            pltpu.VMEM((K,), jnp.int32),
            pltpu.VMEM((K,), jnp.int32),
            pltpu.VMEM((K,), jnp.int32),
            pltpu.VMEM((K, D), jnp.float32),
            pltpu.VMEM((K, D), jnp.float32),
            pltpu.VMEM((K, D), jnp.float32),
            pltpu.VMEM_SHARED((NP, D), jnp.float32),
            pltpu.SemaphoreType.DMA,
            pltpu.SemaphoreType.DMA,
            pltpu.SemaphoreType.DMA,
            pltpu.SemaphoreType.DMA,
            pltpu.SemaphoreType.DMA,
            pltpu.SemaphoreType.DMA,
        ],
    )(_agg_body)
    packed = jnp.bitwise_or(jnp.left_shift(dst, 14), src).reshape(NW, EPW)
    wu = lax.bitcast_convert_type(
        edge_weight.astype(jnp.bfloat16), jnp.uint16
    ).astype(jnp.uint32).reshape(NW, NCHUNK, K)
    wi = jnp.bitwise_or(
        jnp.left_shift(wu[:, :, WC:], 16), wu[:, :, :WC]
    )
    w_st = lax.bitcast_convert_type(wi, jnp.int32).reshape(NW, NCHUNK * WC)
    w_st = jnp.pad(w_st, ((0, 0), (0, WPAD)))
    return f(support, packed, w_st)


def _comb_body(p_ref, x_ref, ws_ref, b_ref, pre_ref, st_ref):
    i = pl.program_id(0)
    v = p_ref[0] + p_ref[1] + b_ref[...] + jnp.dot(
        x_ref[...], ws_ref[...], preferred_element_type=jnp.float32
    )
    pre_ref[...] = v
    cs = jnp.sum(v, axis=0, keepdims=True)
    cs2 = jnp.sum(v * v, axis=0, keepdims=True)
    st = jnp.concatenate([cs, cs2, jnp.zeros((6, D), jnp.float32)], axis=0)

    @pl.when(i == 0)
    def _():
        st_ref[...] = st

    @pl.when(i > 0)
    def _():
        st_ref[...] += st


def _combine(partials, x, self_weight, bias):
    return pl.pallas_call(
        _comb_body,
        grid=(NB,),
        in_specs=[
            pl.BlockSpec((NC, BM, D), lambda i: (0, i, 0)),
            pl.BlockSpec((BM, D), lambda i: (i, 0)),
            pl.BlockSpec((D, D), lambda i: (0, 0)),
            pl.BlockSpec((1, D), lambda i: (0, 0)),
        ],
        out_specs=[
            pl.BlockSpec((BM, D), lambda i: (i, 0)),
            pl.BlockSpec((8, D), lambda i: (0, 0)),
        ],
        out_shape=[
            jax.ShapeDtypeStruct((N, D), jnp.float32),
            jax.ShapeDtypeStruct((8, D), jnp.float32),
        ],
    )(partials, x, self_weight, bias.reshape(1, D))


def _bn_body(pre_ref, st_ref, g_ref, b_ref, o_ref):
    s = st_ref[0:1, :]
    s2 = st_ref[1:2, :]
    mean = s / N
    var = s2 / N - mean * mean
    rstd = lax.rsqrt(var + 1e-5)
    o_ref[...] = (pre_ref[...] - mean) * (rstd * g_ref[...]) + b_ref[...]


def _batchnorm(pre, stats, gamma, beta):
    return pl.pallas_call(
        _bn_body,
        grid=(NB,),
        in_specs=[
            pl.BlockSpec((BM, D), lambda i: (i, 0)),
            pl.BlockSpec((8, D), lambda i: (0, 0)),
            pl.BlockSpec((1, D), lambda i: (0, 0)),
            pl.BlockSpec((1, D), lambda i: (0, 0)),
        ],
        out_specs=pl.BlockSpec((BM, D), lambda i: (i, 0)),
        out_shape=jax.ShapeDtypeStruct((N, D), jnp.float32),
    )(pre, stats, gamma.reshape(1, D), beta.reshape(1, D))


def kernel(x, edge_weight, weight, self_weight, bias, gamma, beta, edge_index):
    support = _support_mm(x, weight)
    dst = edge_index[0]
    src = edge_index[1]
    partials = _aggregate(support, src, dst, edge_weight)
    pre, stats = _combine(partials, x, self_weight, bias)
    return _batchnorm(pre, stats, gamma, beta)


# speedup vs baseline: 3.4171x; 1.0512x over previous
import functools

import jax
import jax.numpy as jnp
from jax import lax
from jax.experimental import pallas as pl
from jax.experimental.pallas import tpu as pltpu
from jax.experimental.pallas import tpu_sc as plsc

N = 10000
E = 320000
D = 128

NC = 2
NS = 16
L = 16
NW = NC * NS
EPW = E // NW
K = 80
NCHUNK = EPW // K
NP = 10240
RPT = NP // NS
ZR = 128
WC = 40
WPAD = 8

BM = 1000
NB = N // BM


ER = E // D
EB = ER // NB


def _mm_body(x_ref, w_ref, e_ref, sup_ref, pk_ref):
    sup_ref[...] = jnp.dot(
        x_ref[...], w_ref[...], preferred_element_type=jnp.float32
    )

    @pl.when(pl.program_id(0) == 0)
    def _():
        pk_ref[...] = jnp.bitwise_or(jnp.left_shift(e_ref[0], 14), e_ref[1])


def _support_mm(x, weight, edge_index):
    return pl.pallas_call(
        _mm_body,
        grid=(NB,),
        in_specs=[
            pl.BlockSpec((BM, D), lambda i: (i, 0)),
            pl.BlockSpec((D, D), lambda i: (0, 0)),
            pl.BlockSpec((2, ER, D), lambda i: (0, 0, 0)),
        ],
        out_specs=[
            pl.BlockSpec((BM, D), lambda i: (i, 0)),
            pl.BlockSpec((ER, D), lambda i: (0, 0)),
        ],
        out_shape=[
            jax.ShapeDtypeStruct((N, D), jnp.float32),
            jax.ShapeDtypeStruct((ER, D), jnp.int32),
        ],
    )(x, weight, edge_index.reshape(2, ER, D))


def _agg_body(sup_hbm, packed_hbm, w_hbm, out_hbm,
              packed_all, w_stage, w_chunk,
              idx0, idx1, idx2, rows0, rows1, rows2,
              acc_sh, gsem0, gsem1, gsem2, ssem0, ssem1, ssem2):
    idx = (idx0, idx1, idx2)
    rows = (rows0, rows1, rows2)
    gsem = (gsem0, gsem1, gsem2)
    ssem = (ssem0, ssem1, ssem2)
    cid = lax.axis_index("c")
    sid = lax.axis_index("s")
    wid = sid * NC + cid

    pltpu.sync_copy(packed_hbm.at[pl.ds(wid * EPW, EPW)], packed_all)
    pltpu.sync_copy(w_hbm.at[wid], w_stage)

    def _zrow(r, _):
        for j in range(D // L):
            rows0[r, pl.ds(j * L, L)] = jnp.zeros((L,), jnp.float32)
        return 0
    lax.fori_loop(0, K, _zrow, 0)
    for k in range(RPT // K):
        pltpu.sync_copy(rows0, acc_sh.at[pl.ds(sid * RPT + k * K, K)])
    plsc.subcore_barrier()

    def _unpack_src(t, b):
        def _g(g, _):
            p = packed_all[pl.ds(t * K + g * L, L)]
            idx[b][pl.ds(g * L, L)] = jnp.bitwise_and(p, 16383)
            return 0
        lax.fori_loop(0, K // L, _g, 0)

    def _unpack_dst(t, b):
        def _g(g, _):
            p = packed_all[pl.ds(t * K + g * L, L)]
            idx[b][pl.ds(g * L, L)] = lax.shift_right_logical(p, 14)
            return 0
        lax.fori_loop(0, K // L, _g, 0)

    def _expand(v):
        wa = lax.bitcast_convert_type(jnp.left_shift(v, 16), jnp.float32)
        wb = lax.bitcast_convert_type(
            jnp.bitwise_and(v, jnp.int32(-65536)), jnp.float32
        )
        return wa, wb

    def _unpack_w(t):
        vt = w_stage[pl.ds(t * WC + 2 * L, L)]
        wa, wb = _expand(vt)
        w_chunk[pl.ds(2 * L, L)] = wa
        w_chunk[pl.ds(72, L)] = wb

        def _g(g, _):
            v = w_stage[pl.ds(t * WC + g * L, L)]
            wa, wb = _expand(v)
            w_chunk[pl.ds(g * L, L)] = wa
            w_chunk[pl.ds(40 + g * L, L)] = wb
            return 0
        lax.fori_loop(0, 2, _g, 0)

    def _scale(b):
        rv = rows[b]

        def _body(g, _):
            w16 = w_chunk[pl.ds(g * L, L)]
            for i in range(L):
                wi = jnp.full((L,), w16[i], jnp.float32)
                e = g * L + i
                for j in range(D // L):
                    rv[e, pl.ds(j * L, L)] = rv[e, pl.ds(j * L, L)] * wi
            return 0
        lax.fori_loop(0, K // L, _body, 0)

    def _issue_gather(t, b):
        _unpack_src(t, b)
        pltpu.async_copy(sup_hbm.at[idx[b]], rows[b], gsem[b])

    def _wait_gather(b):
        pltpu.make_async_copy(sup_hbm.at[idx[b]], rows[b], gsem[b]).wait()

    def _issue_scatter(t, b):
        _unpack_dst(t, b)
        pltpu.async_copy(rows[b], acc_sh.at[idx[b]], ssem[b], add=True)

    def _wait_scatter(b):
        pltpu.make_async_copy(rows[b], acc_sh.at[idx[b]], ssem[b]).wait()

    _issue_gather(0, 0)
    _issue_gather(1, 1)

    def _steady(u, _):
        for i in range(3):
            c = 3 * u + i
            b = i
            bn = (i + 2) % 3
            _wait_gather(b)
            _unpack_w(c)
            _scale(b)
            _issue_scatter(c, b)

            @pl.when(c >= 1)
            def _():
                _wait_scatter(bn)
            _issue_gather(c + 2, bn)
        return 0
    lax.fori_loop(0, (NCHUNK - 2) // 3, _steady, 0)

    for c in (NCHUNK - 2, NCHUNK - 1):
        b = c % 3
        _wait_gather(b)
        _unpack_w(c)
        _scale(b)
        _issue_scatter(c, b)
    for b in ((NCHUNK - 3) % 3, (NCHUNK - 2) % 3, (NCHUNK - 1) % 3):
        _wait_scatter(b)
    plsc.subcore_barrier()

    for k in range(RPT // ZR):
        off = sid * RPT + k * ZR
        pltpu.sync_copy(acc_sh.at[pl.ds(off, ZR)], out_hbm.at[cid, pl.ds(off, ZR)])


def _aggregate(support, packed, edge_weight):
    mesh = plsc.VectorSubcoreMesh(core_axis_name="c", subcore_axis_name="s")
    f = functools.partial(
        pl.kernel,
        mesh=mesh,
        out_type=jax.ShapeDtypeStruct((NC, NP, D), jnp.float32),
        scratch_types=[
            pltpu.VMEM((EPW,), jnp.int32),
            pltpu.VMEM((NCHUNK * WC + WPAD,), jnp.int32),
            pltpu.VMEM((96,), jnp.float32),
            pltpu.VMEM((K,), jnp.int32),
            pltpu.VMEM((K,), jnp.int32),
            pltpu.VMEM((K,), jnp.int32),
            pltpu.VMEM((K, D), jnp.float32),
            pltpu.VMEM((K, D), jnp.float32),
            pltpu.VMEM((K, D), jnp.float32),
            pltpu.VMEM_SHARED((NP, D), jnp.float32),
            pltpu.SemaphoreType.DMA,
            pltpu.SemaphoreType.DMA,
            pltpu.SemaphoreType.DMA,
            pltpu.SemaphoreType.DMA,
            pltpu.SemaphoreType.DMA,
            pltpu.SemaphoreType.DMA,
        ],
    )(_agg_body)
    wu = lax.bitcast_convert_type(
        edge_weight.astype(jnp.bfloat16), jnp.uint16
    ).astype(jnp.uint32).reshape(NW, NCHUNK, K)
    wi = jnp.bitwise_or(
        jnp.left_shift(wu[:, :, WC:], 16), wu[:, :, :WC]
    )
    w_st = lax.bitcast_convert_type(wi, jnp.int32).reshape(NW, NCHUNK * WC)
    w_st = jnp.pad(w_st, ((0, 0), (0, WPAD)))
    return f(support, packed, w_st)


def _comb_body(p_ref, x_ref, ws_ref, b_ref, pre_ref, st_ref):
    i = pl.program_id(0)
    v = p_ref[0] + p_ref[1] + b_ref[...] + jnp.dot(
        x_ref[...], ws_ref[...], preferred_element_type=jnp.float32
    )
    pre_ref[...] = v
    cs = jnp.sum(v, axis=0, keepdims=True)
    cs2 = jnp.sum(v * v, axis=0, keepdims=True)
    st = jnp.concatenate([cs, cs2, jnp.zeros((6, D), jnp.float32)], axis=0)

    @pl.when(i == 0)
    def _():
        st_ref[...] = st

    @pl.when(i > 0)
    def _():
        st_ref[...] += st


def _combine(partials, x, self_weight, bias):
    return pl.pallas_call(
        _comb_body,
        grid=(NB,),
        in_specs=[
            pl.BlockSpec((NC, BM, D), lambda i: (0, i, 0)),
            pl.BlockSpec((BM, D), lambda i: (i, 0)),
            pl.BlockSpec((D, D), lambda i: (0, 0)),
            pl.BlockSpec((1, D), lambda i: (0, 0)),
        ],
        out_specs=[
            pl.BlockSpec((BM, D), lambda i: (i, 0)),
            pl.BlockSpec((8, D), lambda i: (0, 0)),
        ],
        out_shape=[
            jax.ShapeDtypeStruct((N, D), jnp.float32),
            jax.ShapeDtypeStruct((8, D), jnp.float32),
        ],
    )(partials, x, self_weight, bias.reshape(1, D))


def _bn_body(pre_ref, st_ref, g_ref, b_ref, o_ref):
    s = st_ref[0:1, :]
    s2 = st_ref[1:2, :]
    mean = s / N
    var = s2 / N - mean * mean
    rstd = lax.rsqrt(var + 1e-5)
    o_ref[...] = (pre_ref[...] - mean) * (rstd * g_ref[...]) + b_ref[...]


def _batchnorm(pre, stats, gamma, beta):
    return pl.pallas_call(
        _bn_body,
        grid=(NB,),
        in_specs=[
            pl.BlockSpec((BM, D), lambda i: (i, 0)),
            pl.BlockSpec((8, D), lambda i: (0, 0)),
            pl.BlockSpec((1, D), lambda i: (0, 0)),
            pl.BlockSpec((1, D), lambda i: (0, 0)),
        ],
        out_specs=pl.BlockSpec((BM, D), lambda i: (i, 0)),
        out_shape=jax.ShapeDtypeStruct((N, D), jnp.float32),
    )(pre, stats, gamma.reshape(1, D), beta.reshape(1, D))


def kernel(x, edge_weight, weight, self_weight, bias, gamma, beta, edge_index):
    support, packed = _support_mm(x, weight, edge_index)
    partials = _aggregate(support, packed.reshape(E), edge_weight)
    pre, stats = _combine(partials, x, self_weight, bias)
    return _batchnorm(pre, stats, gamma, beta)


# speedup vs baseline: 3.4297x; 1.0037x over previous
import functools

import jax
import jax.numpy as jnp
from jax import lax
from jax.experimental import pallas as pl
from jax.experimental.pallas import tpu as pltpu
from jax.experimental.pallas import tpu_sc as plsc

N = 10000
E = 320000
D = 128

NC = 2
NS = 16
L = 16
NW = NC * NS
EPW = E // NW
K = 80
NCHUNK = EPW // K
NP = 10240
RPT = NP // NS
ZR = 128
WC = 40
WPAD = 8

BM = 1000
NB = N // BM


ER = E // D
EB = ER // NB


def _mm_body(x_ref, w_ref, e_ref, sup_ref, pk_ref):
    sup_ref[...] = jnp.dot(
        x_ref[...], w_ref[...], preferred_element_type=jnp.float32
    )

    @pl.when(pl.program_id(0) == 0)
    def _():
        pk_ref[...] = jnp.bitwise_or(jnp.left_shift(e_ref[0], 14), e_ref[1])


def _support_mm(x, weight, edge_index):
    return pl.pallas_call(
        _mm_body,
        grid=(NB,),
        in_specs=[
            pl.BlockSpec((BM, D), lambda i: (i, 0)),
            pl.BlockSpec((D, D), lambda i: (0, 0)),
            pl.BlockSpec((2, ER, D), lambda i: (0, 0, 0)),
        ],
        out_specs=[
            pl.BlockSpec((BM, D), lambda i: (i, 0)),
            pl.BlockSpec((ER, D), lambda i: (0, 0)),
        ],
        out_shape=[
            jax.ShapeDtypeStruct((N, D), jnp.float32),
            jax.ShapeDtypeStruct((ER, D), jnp.int32),
        ],
    )(x, weight, edge_index.reshape(2, ER, D))


def _agg_body(sup_hbm, packed_hbm, w_hbm, out_hbm,
              packed_all, w_stage, w_chunk,
              idx0, idx1, idx2, idxd0, idxd1, idxd2,
              rows0, rows1, rows2,
              acc_sh, gsem0, gsem1, gsem2, ssem0, ssem1, ssem2):
    idx = (idx0, idx1, idx2)
    idxd = (idxd0, idxd1, idxd2)
    rows = (rows0, rows1, rows2)
    gsem = (gsem0, gsem1, gsem2)
    ssem = (ssem0, ssem1, ssem2)
    cid = lax.axis_index("c")
    sid = lax.axis_index("s")
    wid = sid * NC + cid

    pltpu.sync_copy(packed_hbm.at[pl.ds(wid * EPW, EPW)], packed_all)
    pltpu.sync_copy(w_hbm.at[wid], w_stage)

    def _zrow(r, _):
        for j in range(D // L):
            rows0[r, pl.ds(j * L, L)] = jnp.zeros((L,), jnp.float32)
        return 0
    lax.fori_loop(0, K, _zrow, 0)
    for k in range(RPT // K):
        pltpu.sync_copy(rows0, acc_sh.at[pl.ds(sid * RPT + k * K, K)])
    plsc.subcore_barrier()

    def _unpack_idx(t, b):
        def _g(g, _):
            p = packed_all[pl.ds(t * K + g * L, L)]
            idx[b][pl.ds(g * L, L)] = jnp.bitwise_and(p, 16383)
            idxd[b][pl.ds(g * L, L)] = lax.shift_right_logical(p, 14)
            return 0
        lax.fori_loop(0, K // L, _g, 0)

    def _expand(v):
        wa = lax.bitcast_convert_type(jnp.left_shift(v, 16), jnp.float32)
        wb = lax.bitcast_convert_type(
            jnp.bitwise_and(v, jnp.int32(-65536)), jnp.float32
        )
        return wa, wb

    def _unpack_w(t):
        vt = w_stage[pl.ds(t * WC + 2 * L, L)]
        wa, wb = _expand(vt)
        w_chunk[pl.ds(2 * L, L)] = wa
        w_chunk[pl.ds(72, L)] = wb

        def _g(g, _):
            v = w_stage[pl.ds(t * WC + g * L, L)]
            wa, wb = _expand(v)
            w_chunk[pl.ds(g * L, L)] = wa
            w_chunk[pl.ds(40 + g * L, L)] = wb
            return 0
        lax.fori_loop(0, 2, _g, 0)

    def _scale(b):
        rv = rows[b]

        def _body(g, _):
            w16 = w_chunk[pl.ds(g * L, L)]
            for i in range(L):
                wi = jnp.full((L,), w16[i], jnp.float32)
                e = g * L + i
                for j in range(D // L):
                    rv[e, pl.ds(j * L, L)] = rv[e, pl.ds(j * L, L)] * wi
            return 0
        lax.fori_loop(0, K // L, _body, 0)

    def _issue_gather(t, b):
        _unpack_idx(t, b)
        pltpu.async_copy(sup_hbm.at[idx[b]], rows[b], gsem[b])

    def _wait_gather(b):
        pltpu.make_async_copy(sup_hbm.at[idx[b]], rows[b], gsem[b]).wait()

    def _issue_scatter(t, b):
        pltpu.async_copy(rows[b], acc_sh.at[idxd[b]], ssem[b], add=True)

    def _wait_scatter(b):
        pltpu.make_async_copy(rows[b], acc_sh.at[idxd[b]], ssem[b]).wait()

    _issue_gather(0, 0)
    _issue_gather(1, 1)

    def _steady(u, _):
        for i in range(3):
            c = 3 * u + i
            b = i
            bn = (i + 2) % 3
            _wait_gather(b)
            _unpack_w(c)
            _scale(b)
            _issue_scatter(c, b)

            @pl.when(c >= 1)
            def _():
                _wait_scatter(bn)
            _issue_gather(c + 2, bn)
        return 0
    lax.fori_loop(0, (NCHUNK - 2) // 3, _steady, 0)

    for c in (NCHUNK - 2, NCHUNK - 1):
        b = c % 3
        _wait_gather(b)
        _unpack_w(c)
        _scale(b)
        _issue_scatter(c, b)
    for b in ((NCHUNK - 3) % 3, (NCHUNK - 2) % 3, (NCHUNK - 1) % 3):
        _wait_scatter(b)
    plsc.subcore_barrier()

    for k in range(RPT // ZR):
        off = sid * RPT + k * ZR
        pltpu.sync_copy(acc_sh.at[pl.ds(off, ZR)], out_hbm.at[cid, pl.ds(off, ZR)])


def _aggregate(support, packed, edge_weight):
    mesh = plsc.VectorSubcoreMesh(core_axis_name="c", subcore_axis_name="s")
    f = functools.partial(
        pl.kernel,
        mesh=mesh,
        out_type=jax.ShapeDtypeStruct((NC, NP, D), jnp.float32),
        scratch_types=[
            pltpu.VMEM((EPW,), jnp.int32),
            pltpu.VMEM((NCHUNK * WC + WPAD,), jnp.int32),
            pltpu.VMEM((96,), jnp.float32),
            pltpu.VMEM((K,), jnp.int32),
            pltpu.VMEM((K,), jnp.int32),
            pltpu.VMEM((K,), jnp.int32),
            pltpu.VMEM((K,), jnp.int32),
            pltpu.VMEM((K,), jnp.int32),
            pltpu.VMEM((K,), jnp.int32),
            pltpu.VMEM((K, D), jnp.float32),
            pltpu.VMEM((K, D), jnp.float32),
            pltpu.VMEM((K, D), jnp.float32),
            pltpu.VMEM_SHARED((NP, D), jnp.float32),
            pltpu.SemaphoreType.DMA,
            pltpu.SemaphoreType.DMA,
            pltpu.SemaphoreType.DMA,
            pltpu.SemaphoreType.DMA,
            pltpu.SemaphoreType.DMA,
            pltpu.SemaphoreType.DMA,
        ],
    )(_agg_body)
    wu = lax.bitcast_convert_type(
        edge_weight.astype(jnp.bfloat16), jnp.uint16
    ).astype(jnp.uint32).reshape(NW, NCHUNK, K)
    wi = jnp.bitwise_or(
        jnp.left_shift(wu[:, :, WC:], 16), wu[:, :, :WC]
    )
    w_st = lax.bitcast_convert_type(wi, jnp.int32).reshape(NW, NCHUNK * WC)
    w_st = jnp.pad(w_st, ((0, 0), (0, WPAD)))
    return f(support, packed, w_st)


def _comb_body(p_ref, x_ref, ws_ref, b_ref, pre_ref, st_ref):
    i = pl.program_id(0)
    v = p_ref[0] + p_ref[1] + b_ref[...] + jnp.dot(
        x_ref[...], ws_ref[...], preferred_element_type=jnp.float32
    )
    pre_ref[...] = v
    cs = jnp.sum(v, axis=0, keepdims=True)
    cs2 = jnp.sum(v * v, axis=0, keepdims=True)
    st = jnp.concatenate([cs, cs2, jnp.zeros((6, D), jnp.float32)], axis=0)

    @pl.when(i == 0)
    def _():
        st_ref[...] = st

    @pl.when(i > 0)
    def _():
        st_ref[...] += st


def _combine(partials, x, self_weight, bias):
    return pl.pallas_call(
        _comb_body,
        grid=(NB,),
        in_specs=[
            pl.BlockSpec((NC, BM, D), lambda i: (0, i, 0)),
            pl.BlockSpec((BM, D), lambda i: (i, 0)),
            pl.BlockSpec((D, D), lambda i: (0, 0)),
            pl.BlockSpec((1, D), lambda i: (0, 0)),
        ],
        out_specs=[
            pl.BlockSpec((BM, D), lambda i: (i, 0)),
            pl.BlockSpec((8, D), lambda i: (0, 0)),
        ],
        out_shape=[
            jax.ShapeDtypeStruct((N, D), jnp.float32),
            jax.ShapeDtypeStruct((8, D), jnp.float32),
        ],
    )(partials, x, self_weight, bias.reshape(1, D))


def _bn_body(pre_ref, st_ref, g_ref, b_ref, o_ref):
    s = st_ref[0:1, :]
    s2 = st_ref[1:2, :]
    mean = s / N
    var = s2 / N - mean * mean
    rstd = lax.rsqrt(var + 1e-5)
    o_ref[...] = (pre_ref[...] - mean) * (rstd * g_ref[...]) + b_ref[...]


def _batchnorm(pre, stats, gamma, beta):
    return pl.pallas_call(
        _bn_body,
        grid=(NB,),
        in_specs=[
            pl.BlockSpec((BM, D), lambda i: (i, 0)),
            pl.BlockSpec((8, D), lambda i: (0, 0)),
            pl.BlockSpec((1, D), lambda i: (0, 0)),
            pl.BlockSpec((1, D), lambda i: (0, 0)),
        ],
        out_specs=pl.BlockSpec((BM, D), lambda i: (i, 0)),
        out_shape=jax.ShapeDtypeStruct((N, D), jnp.float32),
    )(pre, stats, gamma.reshape(1, D), beta.reshape(1, D))


def kernel(x, edge_weight, weight, self_weight, bias, gamma, beta, edge_index):
    support, packed = _support_mm(x, weight, edge_index)
    partials = _aggregate(support, packed.reshape(E), edge_weight)
    pre, stats = _combine(partials, x, self_weight, bias)
    return _batchnorm(pre, stats, gamma, beta)
